# baseline (device time: 25471 ns/iter reference)
import os

import numpy as np
import jax
import jax.numpy as jnp
from jax import lax
from jax.experimental import pallas as pl
from jax.experimental.pallas import tpu as pltpu

N_DEV = 4
B, SQ, D = 2, 256, 768
DH = 64
HS = SQ // 2


def kernel(x, Wq, Wk, Wv, Wo):
    hd = Wq.shape[1]
    hloc = hd // DH
    skip_comm = bool(os.environ.get("KERNEL_SKIP_COMM"))

    def body(x_ref, wq_ref, wk_ref, wv_ref, wo_ref,
             out_ref, comm_ref, send_sems, recv_sems):
        my = lax.axis_index("i")
        peer = [my ^ 1, 3 - my]

        barrier_sem = pltpu.get_barrier_semaphore()
        for p in range(2):
            pl.semaphore_signal(
                barrier_sem, inc=1,
                device_id=(peer[p],), device_id_type=pl.DeviceIdType.MESH,
            )
        pl.semaphore_wait(barrier_sem, 2)

        def slot(r, b, half, recv):
            return ((r * 2 + b) * 2 + half) * 2 + recv

        def exchange_start(r, b, half, data_bf16):
            dst = peer[half] if r == 0 else peer[1 - half]
            comm_ref[slot(r, b, half, 0)] = data_bf16
            rdma = pltpu.make_async_remote_copy(
                src_ref=comm_ref.at[slot(r, b, half, 0)],
                dst_ref=comm_ref.at[slot(r, b, half, 1)],
                send_sem=send_sems.at[r, b, half],
                recv_sem=recv_sems.at[r, b, half],
                device_id=(dst,),
                device_id_type=pl.DeviceIdType.MESH,
            )
            rdma.start()
            return rdma

        lane = lax.broadcasted_iota(jnp.int32, (SQ, hd), 1)
        j2 = (lane % DH) // 2 * 2
        inv = jnp.exp(j2.astype(jnp.float32) * (-np.log(10000.0) / DH))
        posr = lax.broadcasted_iota(jnp.int32, (SQ, hd), 0)
        ang = posr.astype(jnp.float32) * inv
        cos_t = jnp.cos(ang)
        sin_t = jnp.sin(ang)
        even = (lane % 2) == 0

        def rope(t):
            rot = jnp.where(even, -jnp.roll(t, -1, axis=1),
                            jnp.roll(t, 1, axis=1))
            return t * cos_t + rot * sin_t

        wq = wq_ref[...].astype(jnp.bfloat16)
        wk = wk_ref[...].astype(jnp.bfloat16)
        wv = wv_ref[...].astype(jnp.bfloat16)
        wo = wo_ref[...].astype(jnp.bfloat16)

        def qkv_rope(b):
            xb = x_ref[b].astype(jnp.bfloat16)
            q = jnp.dot(xb, wq, preferred_element_type=jnp.float32)
            k = jnp.dot(xb, wk, preferred_element_type=jnp.float32)
            v = jnp.dot(xb, wv, preferred_element_type=jnp.float32).astype(
                jnp.bfloat16
            )
            qr = (rope(q) * 0.125).astype(jnp.bfloat16)
            kr = rope(k).astype(jnp.bfloat16)
            return qr, kr, v

        def proj_half(qkv, half):
            qr, kr, v = qkv
            rows = slice(half * HS, (half + 1) * HS)
            ctxs = []
            for h in range(hloc):
                cols = slice(h * DH, (h + 1) * DH)
                s = lax.dot_general(
                    qr[rows, cols], kr[:, cols],
                    (((1,), (1,)), ((), ())),
                    preferred_element_type=jnp.float32,
                )
                e = jnp.exp(s)
                r_inv = 1.0 / jnp.sum(e, axis=-1, keepdims=True)
                ctx = jnp.dot(e.astype(jnp.bfloat16), v[:, cols],
                              preferred_element_type=jnp.float32)
                ctxs.append((ctx * r_inv).astype(jnp.bfloat16))
            ctx = jnp.concatenate(ctxs, axis=-1)
            return jnp.dot(ctx, wo, preferred_element_type=jnp.float32)

        if skip_comm:
            for b in range(B):
                qkv = qkv_rope(b)
                for half in range(2):
                    out_ref[b, half * HS:(half + 1) * HS] = proj_half(
                        qkv, half
                    ).astype(jnp.bfloat16)
            return

        part, acc, r0, r1 = {}, {}, {}, {}

        def round1(b, half):
            r0[b, half].wait()
            acc[b, half] = (
                part[b, half]
                + comm_ref[slot(0, b, half, 1)].astype(jnp.float32)
            )
            r1[b, half] = exchange_start(
                1, b, half, acc[b, half].astype(jnp.bfloat16)
            )

        def finish(b, half):
            r1[b, half].wait()
            out_ref[b, half * HS:(half + 1) * HS] = (
                acc[b, half]
                + comm_ref[slot(1, b, half, 1)].astype(jnp.float32)
            ).astype(jnp.bfloat16)

        qkv0 = qkv_rope(0)
        part[0, 0] = proj_half(qkv0, 0)
        r0[0, 0] = exchange_start(0, 0, 0, part[0, 0].astype(jnp.bfloat16))
        part[0, 1] = proj_half(qkv0, 1)
        r0[0, 1] = exchange_start(0, 0, 1, part[0, 1].astype(jnp.bfloat16))

        qkv1 = qkv_rope(1)
        part[1, 0] = proj_half(qkv1, 0)
        r0[1, 0] = exchange_start(0, 1, 0, part[1, 0].astype(jnp.bfloat16))

        round1(0, 0)
        round1(0, 1)

        part[1, 1] = proj_half(qkv1, 1)
        r0[1, 1] = exchange_start(0, 1, 1, part[1, 1].astype(jnp.bfloat16))

        round1(1, 0)
        finish(0, 0)
        finish(0, 1)
        round1(1, 1)
        finish(1, 0)
        finish(1, 1)

    return pl.pallas_call(
        body,
        out_shape=jax.ShapeDtypeStruct((B, SQ, D), jnp.bfloat16),
        in_specs=[pl.BlockSpec(memory_space=pltpu.VMEM)] * 5,
        out_specs=pl.BlockSpec(memory_space=pltpu.VMEM),
        scratch_shapes=[
            pltpu.VMEM((16, HS, D), jnp.bfloat16),
            pltpu.SemaphoreType.DMA((2, 2, 2)),
            pltpu.SemaphoreType.DMA((2, 2, 2)),
        ],
        compiler_params=pltpu.CompilerParams(collective_id=0),
    )(x, Wq, Wk, Wv, Wo)


# device time: 20026 ns/iter; 1.2719x vs baseline; 1.2719x over previous
import os

import numpy as np
import jax
import jax.numpy as jnp
from jax import lax
from jax.experimental import pallas as pl
from jax.experimental.pallas import tpu as pltpu

N_DEV = 4
B, SQ, D = 2, 256, 768
DH = 64
HS = SQ // 2


def kernel(x, Wq, Wk, Wv, Wo):
    hd = Wq.shape[1]
    hloc = hd // DH
    skip_comm = bool(os.environ.get("KERNEL_SKIP_COMM"))

    def body(x_ref, wq_ref, wk_ref, wv_ref, wo_ref,
             out_ref, comm_ref, send_sems, recv_sems):
        my = lax.axis_index("i")
        peer = [my ^ 1, 3 - my]

        barrier_sem = pltpu.get_barrier_semaphore()
        for p in range(2):
            pl.semaphore_signal(
                barrier_sem, inc=1,
                device_id=(peer[p],), device_id_type=pl.DeviceIdType.MESH,
            )
        pl.semaphore_wait(barrier_sem, 2)

        def slot(r, b, half, dq, recv):
            return (((r * 2 + b) * 2 + half) * 2 + dq) * 2 + recv

        def exchange_start(r, b, half, dq, data_bf16):
            dst = peer[half] if r == 0 else peer[1 - half]
            comm_ref[slot(r, b, half, dq, 0)] = data_bf16
            rdma = pltpu.make_async_remote_copy(
                src_ref=comm_ref.at[slot(r, b, half, dq, 0)],
                dst_ref=comm_ref.at[slot(r, b, half, dq, 1)],
                send_sem=send_sems.at[r, b, half, dq],
                recv_sem=recv_sems.at[r, b, half, dq],
                device_id=(dst,),
                device_id_type=pl.DeviceIdType.MESH,
            )
            rdma.start()
            return rdma

        lane = lax.broadcasted_iota(jnp.int32, (SQ, hd), 1)
        j2 = (lane % DH) // 2 * 2
        inv = jnp.exp(j2.astype(jnp.float32) * (-np.log(10000.0) / DH))
        posr = lax.broadcasted_iota(jnp.int32, (SQ, hd), 0)
        ang = posr.astype(jnp.float32) * inv
        cos_t = jnp.cos(ang)
        sin_t = jnp.sin(ang)
        even = (lane % 2) == 0

        def rope(t):
            rot = jnp.where(even, -jnp.roll(t, -1, axis=1),
                            jnp.roll(t, 1, axis=1))
            return t * cos_t + rot * sin_t

        wq = wq_ref[...]
        wk = wk_ref[...]
        wv = wv_ref[...]
        wo = wo_ref[...]

        def qkv_rope(b):
            xb = x_ref[b].astype(jnp.bfloat16)
            q = jnp.dot(xb, wq, preferred_element_type=jnp.float32)
            k = jnp.dot(xb, wk, preferred_element_type=jnp.float32)
            v = jnp.dot(xb, wv, preferred_element_type=jnp.float32).astype(
                jnp.bfloat16
            )
            qr = (rope(q) * 0.125).astype(jnp.bfloat16)
            kr = rope(k).astype(jnp.bfloat16)
            return qr, kr, v

        def proj_rows(qkv, rows):
            qr, kr, v = qkv
            ctxs = []
            for h in range(hloc):
                cols = slice(h * DH, (h + 1) * DH)
                s = lax.dot_general(
                    qr[rows, cols], kr[:, cols],
                    (((1,), (1,)), ((), ())),
                    preferred_element_type=jnp.float32,
                )
                e = jnp.exp(s.astype(jnp.bfloat16))
                r_inv = 1.0 / jnp.sum(e, axis=-1, keepdims=True,
                                      dtype=jnp.float32)
                ctx = jnp.dot(e, v[:, cols],
                              preferred_element_type=jnp.float32)
                ctxs.append((ctx * r_inv).astype(jnp.bfloat16))
            ctx = jnp.concatenate(ctxs, axis=-1)
            return jnp.dot(ctx, wo, preferred_element_type=jnp.float32)

        def attn_partial(b):
            return proj_rows(qkv_rope(b), slice(0, SQ))

        if skip_comm:
            for b in range(B):
                out_ref[b] = attn_partial(b).astype(jnp.bfloat16)
            return

        part, acc, r0, r1 = {}, {}, {}, {}
        DQ = D // 2

        def round1(b, half, dq):
            r0[b, half, dq].wait()
            acc[b, half, dq] = (
                part[b, half, dq]
                + comm_ref[slot(0, b, half, dq, 1)].astype(jnp.float32)
            )
            r1[b, half, dq] = exchange_start(
                1, b, half, dq, acc[b, half, dq].astype(jnp.bfloat16)
            )

        def finish(b, half, dq):
            r1[b, half, dq].wait()
            out_ref[b, half * HS:(half + 1) * HS,
                    dq * DQ:(dq + 1) * DQ] = (
                acc[b, half, dq]
                + comm_ref[slot(1, b, half, dq, 1)].astype(jnp.float32)
            ).astype(jnp.bfloat16)

        def round0(b):
            p = attn_partial(b)
            for half in range(2):
                for dq in range(2):
                    part[b, half, dq] = p[
                        half * HS:(half + 1) * HS, dq * DQ:(dq + 1) * DQ
                    ]
                    r0[b, half, dq] = exchange_start(
                        0, b, half, dq, part[b, half, dq].astype(jnp.bfloat16)
                    )

        round0(0)
        round0(1)
        for half in range(2):
            for dq in range(2):
                round1(0, half, dq)
        for dq in range(2):
            for half in range(2):
                round1(1, half, dq)
        for b in range(B):
            for dq in range(2):
                for half in range(2):
                    finish(b, half, dq)

    return pl.pallas_call(
        body,
        out_shape=jax.ShapeDtypeStruct((B, SQ, D), jnp.bfloat16),
        in_specs=[pl.BlockSpec(memory_space=pltpu.VMEM)] * 5,
        out_specs=pl.BlockSpec(memory_space=pltpu.VMEM),
        scratch_shapes=[
            pltpu.VMEM((32, HS, D // 2), jnp.bfloat16),
            pltpu.SemaphoreType.DMA((2, 2, 2, 2)),
            pltpu.SemaphoreType.DMA((2, 2, 2, 2)),
        ],
        compiler_params=pltpu.CompilerParams(collective_id=0),
    )(
        x,
        Wq.astype(jnp.bfloat16),
        Wk.astype(jnp.bfloat16),
        Wv.astype(jnp.bfloat16),
        Wo.astype(jnp.bfloat16),
    )
